# baseline (speedup 1.0000x reference)
import functools

import jax
import jax.numpy as jnp
from jax import lax
from jax.experimental import pallas as pl
from jax.experimental.pallas import tpu as pltpu
from jax.experimental.pallas import tpu_sc as plsc

BATCH = 16384
FIELDS = 26
DIM = 32
NEMB = 1000000
NUM_CORES = 2
NUM_SUBCORES = 16
NW = NUM_CORES * NUM_SUBCORES
NBLK = FIELDS * (BATCH // 128)
BLK_PER_W = NBLK // NW
TB = BATCH // 128
NBUF = 2

FOLD_R = 8192
FOLD_STEPS = -(-NEMB // FOLD_R)

_mesh = plsc.VectorSubcoreMesh(core_axis_name="c", subcore_axis_name="s")


@functools.partial(
    pl.kernel,
    mesh=_mesh,
    out_type=jax.ShapeDtypeStruct((FIELDS, 4, TB, 8, 128), jnp.float32),
    scratch_types=[
        pltpu.VMEM((BLK_PER_W // 2, 256), jnp.int32),
        [pltpu.VMEM((256, DIM), jnp.float32) for _ in range(NBUF)],
        [[pltpu.VMEM((DIM, 129), jnp.float32) for _ in range(2)] for _ in range(NBUF)],
        [pltpu.SemaphoreType.DMA for _ in range(NBUF)],
        [pltpu.SemaphoreType.DMA for _ in range(NBUF)],
    ],
    compiler_params=pltpu.CompilerParams(
        use_tc_tiling_on_sc=False, needs_layout_passes=False
    ),
)
def _gather_all(idx_hbm, table_hbm, out_hbm, idx_v, bufs, tbufs, gsems, wsems):
    wid = lax.axis_index("s") * NUM_CORES + lax.axis_index("c")
    base = wid * BLK_PER_W
    pltpu.sync_copy(idx_hbm.at[pl.ds(wid * (BLK_PER_W // 2), BLK_PER_W // 2)], idx_v)

    col_ids = [lax.iota(jnp.int32, 16) + 16 * h for h in range(2)]

    NPAIR = BLK_PER_W // 2

    def fire_gather(p, b):
        pltpu.async_copy(table_hbm.at[idx_v.at[p]], bufs[b], gsems[b])

    def wait_gather(b):
        pltpu.make_async_copy(table_hbm.at[idx_v.at[0]], bufs[b], gsems[b]).wait()

    def transpose_pair(b):
        buf = bufs[b]
        for half in range(2):
            tbuf = tbufs[b][half]
            for r in range(128):
                row = jnp.full((16,), r, jnp.int32)
                for h in range(2):
                    vals = buf[128 * half + r, pl.ds(16 * h, 16)]
                    plsc.store_scatter(tbuf, [col_ids[h], row], vals)

    def fire_writes(p, b):
        blk = base + 2 * p
        f = blk // TB
        tb = blk % TB
        for half in range(2):
            for tc in range(4):
                pltpu.async_copy(
                    tbufs[b][half].at[pl.ds(8 * tc, 8), pl.ds(0, 128)],
                    out_hbm.at[f, tc, tb + half],
                    wsems[b],
                )

    def wait_writes(b):
        for half in range(2):
            for tc in range(4):
                pltpu.make_async_copy(
                    tbufs[b][half].at[pl.ds(8 * tc, 8), pl.ds(0, 128)],
                    out_hbm.at[0, 0, 0],
                    wsems[b],
                ).wait()

    for b in range(NBUF):
        fire_gather(b, b)

    def body(k3, carry):
        for b in range(NBUF):
            p = NBUF * k3 + b
            wait_gather(b)

            @pl.when(k3 > 0)
            def _():
                wait_writes(b)

            transpose_pair(b)

            @pl.when(p + NBUF < NPAIR)
            def _():
                fire_gather(p + NBUF, b)

            fire_writes(p, b)
        return carry

    lax.fori_loop(0, NPAIR // NBUF, body, 0, unroll=False)
    for b in range(NBUF):
        wait_writes(b)


def _fold_body(x_ref, y_ref):
    xt = x_ref[...].T.reshape(FOLD_R // 4, 4, DIM)
    for u in range(4):
        y_ref[:, DIM * u : DIM * (u + 1)] = xt[:, u, :]


_fold = pl.pallas_call(
    _fold_body,
    grid=(FOLD_STEPS,),
    in_specs=[pl.BlockSpec((DIM, FOLD_R), lambda i: (0, i))],
    out_specs=pl.BlockSpec((FOLD_R // 4, 128), lambda i: (i, 0)),
    out_shape=jax.ShapeDtypeStruct((NEMB // 4, 128), jnp.float32),
)


def kernel(input, embedding):
    idx2d = input.T.reshape(NBLK // 2, 256)
    table_lin = _fold(embedding.T).reshape(NEMB, DIM)
    a5 = _gather_all(idx2d, table_lin)
    return a5.transpose(2, 4, 0, 1, 3).reshape(BATCH, FIELDS, DIM)


# speedup vs baseline: 1.0662x; 1.0662x over previous
import functools

import jax
import jax.numpy as jnp
from jax import lax
from jax.experimental import pallas as pl
from jax.experimental.pallas import tpu as pltpu
from jax.experimental.pallas import tpu_sc as plsc

BATCH = 16384
FIELDS = 26
DIM = 32
NEMB = 1000000
NUM_CORES = 2
NUM_SUBCORES = 16
NW = NUM_CORES * NUM_SUBCORES
NBLK = FIELDS * (BATCH // 128)
BLK_PER_W = NBLK // NW
TB = BATCH // 128
NBUF = 2

FOLD_R = 8192
FOLD_STEPS = -(-NEMB // FOLD_R)

_mesh = plsc.VectorSubcoreMesh(core_axis_name="c", subcore_axis_name="s")


@functools.partial(
    pl.kernel,
    mesh=_mesh,
    out_type=jax.ShapeDtypeStruct((FIELDS, 4, TB, 8, 128), jnp.float32),
    scratch_types=[
        pltpu.VMEM((BLK_PER_W, 128), jnp.int32),
        [pltpu.VMEM((128, DIM), jnp.float32) for _ in range(NBUF)],
        [pltpu.VMEM((DIM, 129), jnp.float32) for _ in range(NBUF)],
        [pltpu.SemaphoreType.DMA for _ in range(NBUF)],
        [pltpu.SemaphoreType.DMA for _ in range(NBUF)],
    ],
    compiler_params=pltpu.CompilerParams(
        use_tc_tiling_on_sc=False, needs_layout_passes=False
    ),
)
def _gather_all(idx_hbm, table_hbm, out_hbm, idx_v, bufs, tbufs, gsems, wsems):
    wid = lax.axis_index("s") * NUM_CORES + lax.axis_index("c")
    base = wid * BLK_PER_W
    pltpu.sync_copy(idx_hbm.at[pl.ds(base, BLK_PER_W)], idx_v)

    col_ids = [lax.iota(jnp.int32, 16) + 16 * h for h in range(2)]

    def fire_gather(k, b):
        pltpu.async_copy(table_hbm.at[idx_v.at[k]], bufs[b], gsems[b])

    def wait_gather(b):
        pltpu.make_async_copy(table_hbm.at[idx_v.at[0]], bufs[b], gsems[b]).wait()

    def transpose_block(b):
        buf, tbuf = bufs[b], tbufs[b]
        for r in range(128):
            row = jnp.full((16,), r, jnp.int32)
            for h in range(2):
                vals = buf[r, pl.ds(16 * h, 16)]
                plsc.store_scatter(tbuf, [col_ids[h], row], vals)

    def fire_writes(k, b):
        blk = base + k
        f = blk // TB
        tb = blk % TB
        for tc in range(4):
            pltpu.async_copy(
                tbufs[b].at[pl.ds(8 * tc, 8), pl.ds(0, 128)],
                out_hbm.at[f, tc, tb],
                wsems[b],
            )

    def wait_writes(b):
        for tc in range(4):
            pltpu.make_async_copy(
                tbufs[b].at[pl.ds(8 * tc, 8), pl.ds(0, 128)],
                out_hbm.at[0, 0, 0],
                wsems[b],
            ).wait()

    for b in range(NBUF):
        fire_gather(b, b)

    def body(k2, carry):
        for b in range(NBUF):
            k = NBUF * k2 + b
            wait_gather(b)

            @pl.when(k2 > 0)
            def _():
                wait_writes(b)

            transpose_block(b)

            @pl.when(k + NBUF < BLK_PER_W)
            def _():
                fire_gather(k + NBUF, b)

            fire_writes(k, b)
        return carry

    lax.fori_loop(0, BLK_PER_W // NBUF, body, 0, unroll=False)
    for b in range(NBUF):
        wait_writes(b)


def _fold_body(x_ref, y_ref):
    xt = x_ref[...].T.reshape(FOLD_R // 4, 4, DIM)
    for u in range(4):
        y_ref[:, DIM * u : DIM * (u + 1)] = xt[:, u, :]


_fold = pl.pallas_call(
    _fold_body,
    grid=(FOLD_STEPS,),
    in_specs=[pl.BlockSpec((DIM, FOLD_R), lambda i: (0, i))],
    out_specs=pl.BlockSpec((FOLD_R // 4, 128), lambda i: (i, 0)),
    out_shape=jax.ShapeDtypeStruct((NEMB // 4, 128), jnp.float32),
)


def kernel(input, embedding):
    idx2d = input.T.reshape(NBLK, 128)
    table_lin = _fold(embedding.T).reshape(NEMB, DIM)
    a5 = _gather_all(idx2d, table_lin)
    return a5.transpose(2, 4, 0, 1, 3).reshape(BATCH, FIELDS, DIM)
